# Initial kernel scaffold; baseline (speedup 1.0000x reference)
import functools

import jax
import jax.numpy as jnp
from jax import lax
from jax.experimental import pallas as pl
from jax.experimental.pallas import tpu as pltpu
from jax.experimental.pallas import tpu_sc as plsc

NG = 100000
MN = 512
KN = 8
HH = 512
WW = 512
HWPIX = HH * WW

BLK = 1024
NPAD = 100352
GRID = NPAD // BLK

NCORES = 2
NSUB = 16
NTILES = NCORES * NSUB
PPT = HWPIX // NTILES
CHUNK = 2048
NCHUNK = NPAD // CHUNK
LANES = 16


def _dist_block(meansT, nodes):
    mnorm = jnp.sum(meansT * meansT, axis=0, keepdims=True)
    nnorm = jnp.sum(nodes * nodes, axis=1, keepdims=True)
    d2 = nnorm + mnorm - 2.0 * jnp.dot(nodes, meansT, preferred_element_type=jnp.float32)
    return jnp.maximum(jnp.sqrt(jnp.maximum(d2, 0.0)), 1e-6)


def _p1_body(meansT_ref, nodes_ref, v8_ref, sum_ref):
    pid = pl.program_id(0)
    dist = _dist_block(meansT_ref[...], nodes_ref[...])
    d = dist
    s8 = jnp.zeros((1, BLK), jnp.float32)
    m = None
    for _ in range(KN):
        m = jnp.min(d, axis=0, keepdims=True)
        s8 = s8 + m
        d = jnp.where(d == m, jnp.float32(jnp.inf), d)
    v8_ref[...] = m
    col = pid * BLK + lax.broadcasted_iota(jnp.int32, (1, BLK), 1)
    total = jnp.sum(jnp.where(col < NG, s8, 0.0))
    @pl.when(pid == 0)
    def _():
        sum_ref[0, 0] = total
    @pl.when(pid != 0)
    def _():
        sum_ref[0, 0] = sum_ref[0, 0] + total


def _p2_body(meansT_ref, v8_ref, sum_ref, nodes_ref, offT_ref, clogT_ref,
             olog_ref, lsc_ref, intr_ref, w2c_ref,
             pix_ref, c0_ref, c1_ref, c2_ref, wv_ref):
    pid = pl.program_id(0)
    meansT = meansT_ref[...]
    dist = _dist_block(meansT, nodes_ref[...])
    tau = sum_ref[0, 0] / jnp.float32(NG * KN) + 1e-6
    v8 = v8_ref[...]
    msk = dist <= v8
    v1 = jnp.min(dist, axis=0, keepdims=True)
    e = jnp.where(msk, jnp.exp((v1 - dist) / tau), 0.0)
    wn = e / jnp.sum(e, axis=0, keepdims=True)
    motion = jnp.dot(offT_ref[...], wn, preferred_element_type=jnp.float32)
    mt = meansT + motion
    w2c = w2c_ref[...]
    R = w2c[0:3, 0:3]
    t = w2c[0:3, 3:4]
    pts = jnp.dot(R, mt, preferred_element_type=jnp.float32) + t
    uvw = jnp.dot(intr_ref[...], pts, preferred_element_type=jnp.float32)
    z = jnp.maximum(uvw[2:3, :], 1e-3)
    u = uvw[0:1, :] / z
    v = uvw[1:2, :] / z
    ui = jnp.clip(jnp.round(u), 0.0, WW - 1).astype(jnp.int32)
    vi = jnp.clip(jnp.round(v), 0.0, HH - 1).astype(jnp.int32)
    pix = vi * WW + ui
    opac = jax.nn.sigmoid(olog_ref[...])
    scale = jnp.exp(lsc_ref[...])
    wgt = opac * scale / (z * z)
    col = pid * BLK + lax.broadcasted_iota(jnp.int32, (1, BLK), 1)
    valid = col < NG
    wgt = jnp.where(valid, wgt, 0.0)
    pix_ref[...] = jnp.where(valid, pix, 0)
    c = jax.nn.sigmoid(clogT_ref[...])
    c0_ref[...] = c[0:1, :] * wgt
    c1_ref[...] = c[1:2, :] * wgt
    c2_ref[...] = c[2:3, :] * wgt
    wv_ref[...] = wgt


def _sc_body(pix_hbm, c0_hbm, c1_hbm, c2_hbm, wv_hbm, zeros_hbm, out_hbm,
             pixb, c0b, c1b, c2b, wvb, acc0, acc1, acc2, accw):
    wid = lax.axis_index("s") * NCORES + lax.axis_index("c")
    base = wid * PPT
    pltpu.sync_copy(zeros_hbm, acc0)
    pltpu.sync_copy(zeros_hbm, acc1)
    pltpu.sync_copy(zeros_hbm, acc2)
    pltpu.sync_copy(zeros_hbm, accw)

    def chunk_body(ci, carry):
        off = ci * CHUNK
        pltpu.sync_copy(pix_hbm.at[pl.ds(off, CHUNK)], pixb)
        pltpu.sync_copy(c0_hbm.at[pl.ds(off, CHUNK)], c0b)
        pltpu.sync_copy(c1_hbm.at[pl.ds(off, CHUNK)], c1b)
        pltpu.sync_copy(c2_hbm.at[pl.ds(off, CHUNK)], c2b)
        pltpu.sync_copy(wv_hbm.at[pl.ds(off, CHUNK)], wvb)

        def grp(g, carry2):
            s = g * LANES
            pv = pixb[pl.ds(s, LANES)]
            lv = pv - base
            mk = (lv >= 0) & (lv < PPT)
            ls = jnp.where(mk, lv, 0)
            plsc.addupdate_scatter(acc0, [ls], c0b[pl.ds(s, LANES)], mask=mk)
            plsc.addupdate_scatter(acc1, [ls], c1b[pl.ds(s, LANES)], mask=mk)
            plsc.addupdate_scatter(acc2, [ls], c2b[pl.ds(s, LANES)], mask=mk)
            plsc.addupdate_scatter(accw, [ls], wvb[pl.ds(s, LANES)], mask=mk)
            return carry2

        return lax.fori_loop(0, CHUNK // LANES, grp, carry)

    lax.fori_loop(0, NCHUNK, chunk_body, 0)

    def norm(g, carry):
        s = g * LANES
        ws = accw[pl.ds(s, LANES)]
        alpha = jnp.clip(ws, 0.0, 1.0)
        sc = alpha / (ws + 1e-6)
        acc0[pl.ds(s, LANES)] = acc0[pl.ds(s, LANES)] * sc
        acc1[pl.ds(s, LANES)] = acc1[pl.ds(s, LANES)] * sc
        acc2[pl.ds(s, LANES)] = acc2[pl.ds(s, LANES)] * sc
        return carry

    lax.fori_loop(0, PPT // LANES, norm, 0)
    pltpu.sync_copy(acc0, out_hbm.at[0, pl.ds(base, PPT)])
    pltpu.sync_copy(acc1, out_hbm.at[1, pl.ds(base, PPT)])
    pltpu.sync_copy(acc2, out_hbm.at[2, pl.ds(base, PPT)])


def kernel(means, log_scales, color_logits, opacity_logits, node_positions,
           node_offsets, intrinsics, world_to_camera, time_index):
    t = jnp.asarray(time_index)
    offsets_t = lax.dynamic_index_in_dim(node_offsets, t, 0, keepdims=False)
    offT = jnp.transpose(offsets_t)

    pad = NPAD - NG
    meansT = jnp.pad(jnp.transpose(means), ((0, 0), (0, pad)))
    clogT = jnp.pad(jnp.transpose(color_logits), ((0, 0), (0, pad)))
    olog = jnp.pad(jnp.transpose(opacity_logits), ((0, 0), (0, pad)))
    lsc = jnp.pad(jnp.transpose(log_scales), ((0, 0), (0, pad)))

    v8, s8 = pl.pallas_call(
        _p1_body,
        grid=(GRID,),
        in_specs=[
            pl.BlockSpec((3, BLK), lambda i: (0, i)),
            pl.BlockSpec((MN, 3), lambda i: (0, 0)),
        ],
        out_specs=[
            pl.BlockSpec((1, BLK), lambda i: (0, i)),
            pl.BlockSpec(memory_space=pltpu.SMEM),
        ],
        out_shape=[
            jax.ShapeDtypeStruct((1, NPAD), jnp.float32),
            jax.ShapeDtypeStruct((1, 1), jnp.float32),
        ],
        compiler_params=pltpu.CompilerParams(
            dimension_semantics=("arbitrary",)),
    )(meansT, node_positions)

    pix, c0, c1, c2, wv = pl.pallas_call(
        _p2_body,
        grid=(GRID,),
        in_specs=[
            pl.BlockSpec((3, BLK), lambda i: (0, i)),
            pl.BlockSpec((1, BLK), lambda i: (0, i)),
            pl.BlockSpec(memory_space=pltpu.SMEM),
            pl.BlockSpec((MN, 3), lambda i: (0, 0)),
            pl.BlockSpec((3, MN), lambda i: (0, 0)),
            pl.BlockSpec((3, BLK), lambda i: (0, i)),
            pl.BlockSpec((1, BLK), lambda i: (0, i)),
            pl.BlockSpec((1, BLK), lambda i: (0, i)),
            pl.BlockSpec((3, 3), lambda i: (0, 0)),
            pl.BlockSpec((4, 4), lambda i: (0, 0)),
        ],
        out_specs=[
            pl.BlockSpec((1, BLK), lambda i: (0, i)),
            pl.BlockSpec((1, BLK), lambda i: (0, i)),
            pl.BlockSpec((1, BLK), lambda i: (0, i)),
            pl.BlockSpec((1, BLK), lambda i: (0, i)),
            pl.BlockSpec((1, BLK), lambda i: (0, i)),
        ],
        out_shape=[
            jax.ShapeDtypeStruct((1, NPAD), jnp.int32),
            jax.ShapeDtypeStruct((1, NPAD), jnp.float32),
            jax.ShapeDtypeStruct((1, NPAD), jnp.float32),
            jax.ShapeDtypeStruct((1, NPAD), jnp.float32),
            jax.ShapeDtypeStruct((1, NPAD), jnp.float32),
        ],
        compiler_params=pltpu.CompilerParams(
            dimension_semantics=("arbitrary",)),
    )(meansT, v8, s8, node_positions, offT, clogT, olog, lsc,
      intrinsics, world_to_camera)

    mesh = plsc.VectorSubcoreMesh(core_axis_name="c", subcore_axis_name="s")
    scatter = pl.kernel(
        _sc_body,
        out_type=jax.ShapeDtypeStruct((3, HWPIX), jnp.float32),
        mesh=mesh,
        scratch_types=[
            pltpu.VMEM((CHUNK,), jnp.int32),
            pltpu.VMEM((CHUNK,), jnp.float32),
            pltpu.VMEM((CHUNK,), jnp.float32),
            pltpu.VMEM((CHUNK,), jnp.float32),
            pltpu.VMEM((CHUNK,), jnp.float32),
            pltpu.VMEM((PPT,), jnp.float32),
            pltpu.VMEM((PPT,), jnp.float32),
            pltpu.VMEM((PPT,), jnp.float32),
            pltpu.VMEM((PPT,), jnp.float32),
        ],
    )
    zeros_img = jnp.zeros((PPT,), jnp.float32)
    out = scatter(pix.reshape(NPAD), c0.reshape(NPAD), c1.reshape(NPAD),
                  c2.reshape(NPAD), wv.reshape(NPAD), zeros_img)
    return jnp.transpose(out).reshape(HH, WW, 3)


# speedup vs baseline: 8.7133x; 8.7133x over previous
import functools

import jax
import jax.numpy as jnp
from jax import lax
from jax.experimental import pallas as pl
from jax.experimental.pallas import tpu as pltpu
from jax.experimental.pallas import tpu_sc as plsc

NG = 100000
MN = 512
KN = 8
HH = 512
WW = 512
HWPIX = HH * WW

BLK = 1024
NPAD = 100352
GRID = NPAD // BLK

NCORES = 2
NSUB = 16
NTILES = NCORES * NSUB
PPT = HWPIX // NTILES
CHUNK = 2048
NCHUNK = NPAD // CHUNK
LANES = 16


def _dist_block(meansT, nodes):
    mnorm = jnp.sum(meansT * meansT, axis=0, keepdims=True)
    nnorm = jnp.sum(nodes * nodes, axis=1, keepdims=True)
    dot = jnp.dot(nodes, meansT, preferred_element_type=jnp.float32)
    d2 = (mnorm + nnorm) - 2.0 * dot
    return jnp.maximum(jnp.sqrt(jnp.maximum(d2, 0.0)), 1e-6)


def _p1_body(meansT_ref, nodes_ref, v8_ref, sum_ref):
    pid = pl.program_id(0)
    dist = _dist_block(meansT_ref[...], nodes_ref[...])
    d = dist
    s8 = jnp.zeros((1, BLK), jnp.float32)
    riota = lax.broadcasted_iota(jnp.int32, (MN, BLK), 0)
    m = None
    for _ in range(KN):
        m = jnp.min(d, axis=0, keepdims=True)
        s8 = s8 + m
        fidx = jnp.min(jnp.where(d == m, riota, MN), axis=0, keepdims=True)
        d = jnp.where(riota == fidx, jnp.float32(jnp.inf), d)
    v8_ref[...] = m
    col = pid * BLK + lax.broadcasted_iota(jnp.int32, (1, BLK), 1)
    total = jnp.sum(jnp.where(col < NG, s8, 0.0))
    @pl.when(pid == 0)
    def _():
        sum_ref[0, 0] = total
    @pl.when(pid != 0)
    def _():
        sum_ref[0, 0] = sum_ref[0, 0] + total


def _p2_body(meansT_ref, v8_ref, sum_ref, nodes_ref, offT_ref, clogT_ref,
             olog_ref, lsc_ref, intr_ref, w2c_ref,
             pix_ref, c0_ref, c1_ref, c2_ref, wv_ref):
    pid = pl.program_id(0)
    meansT = meansT_ref[...]
    dist = _dist_block(meansT, nodes_ref[...])
    tau = sum_ref[0, 0] / jnp.float32(NG * KN) + 1e-6
    d = dist
    riota = lax.broadcasted_iota(jnp.int32, (MN, BLK), 0)
    msk = jnp.zeros((MN, BLK), jnp.bool_)
    v1 = None
    for k in range(KN):
        m = jnp.min(d, axis=0, keepdims=True)
        if k == 0:
            v1 = m
        fidx = jnp.min(jnp.where(d == m, riota, MN), axis=0, keepdims=True)
        sel = riota == fidx
        msk = msk | sel
        d = jnp.where(sel, jnp.float32(jnp.inf), d)
    e = jnp.where(msk, jnp.exp((v1 - dist) / tau), 0.0)
    wn = e / jnp.sum(e, axis=0, keepdims=True)
    off = offT_ref[...]
    mo0 = jnp.sum(wn * off[:, 0:1], axis=0, keepdims=True)
    mo1 = jnp.sum(wn * off[:, 1:2], axis=0, keepdims=True)
    mo2 = jnp.sum(wn * off[:, 2:3], axis=0, keepdims=True)
    mt = jnp.concatenate(
        [meansT[0:1, :] + mo0, meansT[1:2, :] + mo1, meansT[2:3, :] + mo2],
        axis=0)
    w2c = w2c_ref[...]
    R = w2c[0:3, 0:3]
    t = w2c[0:3, 3:4]
    pts = jnp.dot(R, mt, preferred_element_type=jnp.float32) + t
    uvw = jnp.dot(intr_ref[...], pts,
                  preferred_element_type=jnp.float32)
    z = jnp.maximum(uvw[2:3, :], 1e-3)
    u = uvw[0:1, :] / z
    v = uvw[1:2, :] / z
    ui = jnp.clip(jnp.round(u), 0.0, WW - 1).astype(jnp.int32)
    vi = jnp.clip(jnp.round(v), 0.0, HH - 1).astype(jnp.int32)
    pix = vi * WW + ui
    opac = jax.nn.sigmoid(olog_ref[...])
    scale = jnp.exp(lsc_ref[...])
    wgt = opac * scale / (z * z)
    col = pid * BLK + lax.broadcasted_iota(jnp.int32, (1, BLK), 1)
    valid = col < NG
    wgt = jnp.where(valid, wgt, 0.0)
    pix_ref[...] = jnp.where(valid, pix, 0)
    c = jax.nn.sigmoid(clogT_ref[...])
    c0_ref[...] = c[0:1, :] * wgt
    c1_ref[...] = c[1:2, :] * wgt
    c2_ref[...] = c[2:3, :] * wgt
    wv_ref[...] = wgt


def _sc_body(pix_hbm, c0_hbm, c1_hbm, c2_hbm, wv_hbm, out_hbm,
             pixb, c0b, c1b, c2b, wvb, acc0, acc1, acc2, accw,
             st0, st1, st2):
    wid = lax.axis_index("s") * NCORES + lax.axis_index("c")
    base = wid * PPT

    def zero(g, carry):
        s = g * LANES
        z = jnp.zeros((LANES,), jnp.float32)
        acc0[pl.ds(s, LANES)] = z
        acc1[pl.ds(s, LANES)] = z
        acc2[pl.ds(s, LANES)] = z
        accw[pl.ds(s, LANES)] = z
        return carry

    lax.fori_loop(0, PPT // LANES, zero, 0)

    def chunk_body(ci, carry):
        pltpu.sync_copy(pix_hbm.at[pl.ds(ci, 1), :], pixb)
        pltpu.sync_copy(c0_hbm.at[pl.ds(ci, 1), :], c0b)
        pltpu.sync_copy(c1_hbm.at[pl.ds(ci, 1), :], c1b)
        pltpu.sync_copy(c2_hbm.at[pl.ds(ci, 1), :], c2b)
        pltpu.sync_copy(wv_hbm.at[pl.ds(ci, 1), :], wvb)

        def grp(g, carry2):
            s = g * LANES
            pv = pixb[0, pl.ds(s, LANES)]
            lv = pv - base
            mk = (lv >= 0) & (lv < PPT)
            ls = jnp.where(mk, lv, 0)
            plsc.addupdate_scatter(acc0, [ls], c0b[0, pl.ds(s, LANES)], mask=mk)
            plsc.addupdate_scatter(acc1, [ls], c1b[0, pl.ds(s, LANES)], mask=mk)
            plsc.addupdate_scatter(acc2, [ls], c2b[0, pl.ds(s, LANES)], mask=mk)
            plsc.addupdate_scatter(accw, [ls], wvb[0, pl.ds(s, LANES)], mask=mk)
            return carry2

        return lax.fori_loop(0, CHUNK // LANES, grp, carry)

    lax.fori_loop(0, NCHUNK, chunk_body, 0)

    def norm(g, carry):
        s = g * LANES
        ws = accw[pl.ds(s, LANES)]
        alpha = jnp.clip(ws, 0.0, 1.0)
        sc = alpha / (ws + 1e-6)
        st0[0, pl.ds(s, LANES)] = acc0[pl.ds(s, LANES)] * sc
        st1[0, pl.ds(s, LANES)] = acc1[pl.ds(s, LANES)] * sc
        st2[0, pl.ds(s, LANES)] = acc2[pl.ds(s, LANES)] * sc
        return carry

    lax.fori_loop(0, PPT // LANES, norm, 0)
    pltpu.sync_copy(st0, out_hbm.at[pl.ds(0, 1), pl.ds(base, PPT)])
    pltpu.sync_copy(st1, out_hbm.at[pl.ds(1, 1), pl.ds(base, PPT)])
    pltpu.sync_copy(st2, out_hbm.at[pl.ds(2, 1), pl.ds(base, PPT)])


def kernel(means, log_scales, color_logits, opacity_logits, node_positions,
           node_offsets, intrinsics, world_to_camera, time_index):
    t = jnp.asarray(time_index)
    offsets_t = lax.dynamic_index_in_dim(node_offsets, t, 0, keepdims=False)

    pad = NPAD - NG
    meansT = jnp.pad(jnp.transpose(means), ((0, 0), (0, pad)))
    clogT = jnp.pad(jnp.transpose(color_logits), ((0, 0), (0, pad)))
    olog = jnp.pad(jnp.transpose(opacity_logits), ((0, 0), (0, pad)))
    lsc = jnp.pad(jnp.transpose(log_scales), ((0, 0), (0, pad)))

    v8, s8 = pl.pallas_call(
        _p1_body,
        grid=(GRID,),
        in_specs=[
            pl.BlockSpec((3, BLK), lambda i: (0, i)),
            pl.BlockSpec((MN, 3), lambda i: (0, 0)),
        ],
        out_specs=[
            pl.BlockSpec((1, BLK), lambda i: (0, i)),
            pl.BlockSpec(memory_space=pltpu.SMEM),
        ],
        out_shape=[
            jax.ShapeDtypeStruct((1, NPAD), jnp.float32),
            jax.ShapeDtypeStruct((1, 1), jnp.float32),
        ],
        compiler_params=pltpu.CompilerParams(
            dimension_semantics=("arbitrary",)),
    )(meansT, node_positions)

    pix, c0, c1, c2, wv = pl.pallas_call(
        _p2_body,
        grid=(GRID,),
        in_specs=[
            pl.BlockSpec((3, BLK), lambda i: (0, i)),
            pl.BlockSpec((1, BLK), lambda i: (0, i)),
            pl.BlockSpec(memory_space=pltpu.SMEM),
            pl.BlockSpec((MN, 3), lambda i: (0, 0)),
            pl.BlockSpec((MN, 3), lambda i: (0, 0)),
            pl.BlockSpec((3, BLK), lambda i: (0, i)),
            pl.BlockSpec((1, BLK), lambda i: (0, i)),
            pl.BlockSpec((1, BLK), lambda i: (0, i)),
            pl.BlockSpec((3, 3), lambda i: (0, 0)),
            pl.BlockSpec((4, 4), lambda i: (0, 0)),
        ],
        out_specs=[
            pl.BlockSpec((1, BLK), lambda i: (0, i)),
            pl.BlockSpec((1, BLK), lambda i: (0, i)),
            pl.BlockSpec((1, BLK), lambda i: (0, i)),
            pl.BlockSpec((1, BLK), lambda i: (0, i)),
            pl.BlockSpec((1, BLK), lambda i: (0, i)),
        ],
        out_shape=[
            jax.ShapeDtypeStruct((1, NPAD), jnp.int32),
            jax.ShapeDtypeStruct((1, NPAD), jnp.float32),
            jax.ShapeDtypeStruct((1, NPAD), jnp.float32),
            jax.ShapeDtypeStruct((1, NPAD), jnp.float32),
            jax.ShapeDtypeStruct((1, NPAD), jnp.float32),
        ],
        compiler_params=pltpu.CompilerParams(
            dimension_semantics=("arbitrary",)),
    )(meansT, v8, s8, node_positions, offsets_t, clogT, olog, lsc,
      intrinsics, world_to_camera)

    mesh = plsc.VectorSubcoreMesh(core_axis_name="c", subcore_axis_name="s")
    scatter = pl.kernel(
        _sc_body,
        out_type=jax.ShapeDtypeStruct((3, HWPIX), jnp.float32),
        mesh=mesh,
        compiler_params=pltpu.CompilerParams(needs_layout_passes=False),
        scratch_types=[
            pltpu.VMEM((1, CHUNK), jnp.int32),
            pltpu.VMEM((1, CHUNK), jnp.float32),
            pltpu.VMEM((1, CHUNK), jnp.float32),
            pltpu.VMEM((1, CHUNK), jnp.float32),
            pltpu.VMEM((1, CHUNK), jnp.float32),
            pltpu.VMEM((PPT,), jnp.float32),
            pltpu.VMEM((PPT,), jnp.float32),
            pltpu.VMEM((PPT,), jnp.float32),
            pltpu.VMEM((PPT,), jnp.float32),
            pltpu.VMEM((1, PPT), jnp.float32),
            pltpu.VMEM((1, PPT), jnp.float32),
            pltpu.VMEM((1, PPT), jnp.float32),
        ],
    )
    out = scatter(pix.reshape(NCHUNK, CHUNK), c0.reshape(NCHUNK, CHUNK),
                  c1.reshape(NCHUNK, CHUNK), c2.reshape(NCHUNK, CHUNK),
                  wv.reshape(NCHUNK, CHUNK))
    return jnp.transpose(out).reshape(HH, WW, 3)


# speedup vs baseline: 10.5203x; 1.2074x over previous
import functools

import jax
import jax.numpy as jnp
from jax import lax
from jax.experimental import pallas as pl
from jax.experimental.pallas import tpu as pltpu
from jax.experimental.pallas import tpu_sc as plsc

NG = 100000
MN = 512
KN = 8
HH = 512
WW = 512
HWPIX = HH * WW

BLK = 1024
NPAD = 100352
GRID = NPAD // BLK

NCORES = 2
NSUB = 16
NTILES = NCORES * NSUB
PPT = HWPIX // NTILES
CHUNK = 2048
NCHUNK = NPAD // CHUNK
LANES = 16


def _dist_block(meansT, nodes):
    mnorm = jnp.sum(meansT * meansT, axis=0, keepdims=True)
    nnorm = jnp.sum(nodes * nodes, axis=1, keepdims=True)
    dot = jnp.dot(nodes, meansT, preferred_element_type=jnp.float32)
    d2 = (mnorm + nnorm) - 2.0 * dot
    return jnp.maximum(jnp.sqrt(jnp.maximum(d2, 0.0)), 1e-6)


def _p1_body(meansT_ref, nodes_ref, packed_ref, sum_ref):
    pid = pl.program_id(0)
    dist = _dist_block(meansT_ref[...], nodes_ref[...])
    d = dist
    s8 = jnp.zeros((1, BLK), jnp.float32)
    riota = lax.broadcasted_iota(jnp.int32, (MN, BLK), 0)
    msk = jnp.zeros((MN, BLK), jnp.bool_)
    for _ in range(KN):
        m = jnp.min(d, axis=0, keepdims=True)
        s8 = s8 + m
        fidx = jnp.min(jnp.where(d == m, riota, MN), axis=0, keepdims=True)
        sel = riota == fidx
        msk = msk | sel
        d = jnp.where(sel, jnp.float32(jnp.inf), d)
    mi = msk.astype(jnp.int32)
    shifts = lax.broadcasted_iota(jnp.int32, (32, 1), 0)
    words = [jnp.sum(mi[w * 32:(w + 1) * 32, :] << shifts, axis=0, keepdims=True)
             for w in range(MN // 32)]
    packed_ref[...] = jnp.concatenate(words, axis=0)
    col = pid * BLK + lax.broadcasted_iota(jnp.int32, (1, BLK), 1)
    total = jnp.sum(jnp.where(col < NG, s8, 0.0))
    @pl.when(pid == 0)
    def _():
        sum_ref[0, 0] = total
    @pl.when(pid != 0)
    def _():
        sum_ref[0, 0] = sum_ref[0, 0] + total


def _p2_body(meansT_ref, packed_ref, sum_ref, nodes_ref, offT_ref, clogT_ref,
             olog_ref, lsc_ref, intr_ref, w2c_ref,
             pix_ref, c0_ref, c1_ref, c2_ref, wv_ref):
    pid = pl.program_id(0)
    meansT = meansT_ref[...]
    dist = _dist_block(meansT, nodes_ref[...])
    tau = sum_ref[0, 0] / jnp.float32(NG * KN) + 1e-6
    packed = packed_ref[...]
    shifts = lax.broadcasted_iota(jnp.int32, (32, 1), 0)
    parts = [(jnp.broadcast_to(packed[w:w + 1, :], (32, BLK)) >> shifts) & 1
             for w in range(MN // 32)]
    msk = jnp.concatenate(parts, axis=0) == 1
    v1 = jnp.min(jnp.where(msk, dist, jnp.float32(jnp.inf)),
                 axis=0, keepdims=True)
    e = jnp.where(msk, jnp.exp((v1 - dist) / tau), 0.0)
    wn = e / jnp.sum(e, axis=0, keepdims=True)
    off = offT_ref[...]
    mo0 = jnp.sum(wn * off[:, 0:1], axis=0, keepdims=True)
    mo1 = jnp.sum(wn * off[:, 1:2], axis=0, keepdims=True)
    mo2 = jnp.sum(wn * off[:, 2:3], axis=0, keepdims=True)
    mt = jnp.concatenate(
        [meansT[0:1, :] + mo0, meansT[1:2, :] + mo1, meansT[2:3, :] + mo2],
        axis=0)
    w2c = w2c_ref[...]
    R = w2c[0:3, 0:3]
    t = w2c[0:3, 3:4]
    pts = jnp.dot(R, mt, preferred_element_type=jnp.float32) + t
    uvw = jnp.dot(intr_ref[...], pts,
                  preferred_element_type=jnp.float32)
    z = jnp.maximum(uvw[2:3, :], 1e-3)
    u = uvw[0:1, :] / z
    v = uvw[1:2, :] / z
    ui = jnp.clip(jnp.round(u), 0.0, WW - 1).astype(jnp.int32)
    vi = jnp.clip(jnp.round(v), 0.0, HH - 1).astype(jnp.int32)
    pix = vi * WW + ui
    opac = jax.nn.sigmoid(olog_ref[...])
    scale = jnp.exp(lsc_ref[...])
    wgt = opac * scale / (z * z)
    col = pid * BLK + lax.broadcasted_iota(jnp.int32, (1, BLK), 1)
    valid = col < NG
    wgt = jnp.where(valid, wgt, 0.0)
    pix_ref[...] = jnp.where(valid, pix, 0)
    c = jax.nn.sigmoid(clogT_ref[...])
    c0_ref[...] = c[0:1, :] * wgt
    c1_ref[...] = c[1:2, :] * wgt
    c2_ref[...] = c[2:3, :] * wgt
    wv_ref[...] = wgt


def _sc_body(pix_hbm, c0_hbm, c1_hbm, c2_hbm, wv_hbm, out_hbm,
             pixb, c0b, c1b, c2b, wvb, acc0, acc1, acc2, accw,
             st0, st1, st2):
    wid = lax.axis_index("s") * NCORES + lax.axis_index("c")
    base = wid * PPT

    def zero(g, carry):
        s = g * LANES
        z = jnp.zeros((LANES,), jnp.float32)
        acc0[pl.ds(s, LANES)] = z
        acc1[pl.ds(s, LANES)] = z
        acc2[pl.ds(s, LANES)] = z
        accw[pl.ds(s, LANES)] = z
        return carry

    lax.fori_loop(0, PPT // LANES, zero, 0)

    def chunk_body(ci, carry):
        pltpu.sync_copy(pix_hbm.at[pl.ds(ci, 1), :], pixb)
        pltpu.sync_copy(c0_hbm.at[pl.ds(ci, 1), :], c0b)
        pltpu.sync_copy(c1_hbm.at[pl.ds(ci, 1), :], c1b)
        pltpu.sync_copy(c2_hbm.at[pl.ds(ci, 1), :], c2b)
        pltpu.sync_copy(wv_hbm.at[pl.ds(ci, 1), :], wvb)

        def grp(g, carry2):
            s = g * LANES
            pv = pixb[0, pl.ds(s, LANES)]
            lv = pv - base
            mk = (lv >= 0) & (lv < PPT)
            ls = jnp.where(mk, lv, 0)
            plsc.addupdate_scatter(acc0, [ls], c0b[0, pl.ds(s, LANES)], mask=mk)
            plsc.addupdate_scatter(acc1, [ls], c1b[0, pl.ds(s, LANES)], mask=mk)
            plsc.addupdate_scatter(acc2, [ls], c2b[0, pl.ds(s, LANES)], mask=mk)
            plsc.addupdate_scatter(accw, [ls], wvb[0, pl.ds(s, LANES)], mask=mk)
            return carry2

        return lax.fori_loop(0, CHUNK // LANES, grp, carry)

    lax.fori_loop(0, NCHUNK, chunk_body, 0)

    def norm(g, carry):
        s = g * LANES
        ws = accw[pl.ds(s, LANES)]
        alpha = jnp.clip(ws, 0.0, 1.0)
        sc = alpha / (ws + 1e-6)
        st0[0, pl.ds(s, LANES)] = acc0[pl.ds(s, LANES)] * sc
        st1[0, pl.ds(s, LANES)] = acc1[pl.ds(s, LANES)] * sc
        st2[0, pl.ds(s, LANES)] = acc2[pl.ds(s, LANES)] * sc
        return carry

    lax.fori_loop(0, PPT // LANES, norm, 0)
    pltpu.sync_copy(st0, out_hbm.at[pl.ds(0, 1), pl.ds(base, PPT)])
    pltpu.sync_copy(st1, out_hbm.at[pl.ds(1, 1), pl.ds(base, PPT)])
    pltpu.sync_copy(st2, out_hbm.at[pl.ds(2, 1), pl.ds(base, PPT)])


def kernel(means, log_scales, color_logits, opacity_logits, node_positions,
           node_offsets, intrinsics, world_to_camera, time_index):
    t = jnp.asarray(time_index)
    offsets_t = lax.dynamic_index_in_dim(node_offsets, t, 0, keepdims=False)

    pad = NPAD - NG
    meansT = jnp.pad(jnp.transpose(means), ((0, 0), (0, pad)))
    clogT = jnp.pad(jnp.transpose(color_logits), ((0, 0), (0, pad)))
    olog = jnp.pad(jnp.transpose(opacity_logits), ((0, 0), (0, pad)))
    lsc = jnp.pad(jnp.transpose(log_scales), ((0, 0), (0, pad)))

    packed, s8 = pl.pallas_call(
        _p1_body,
        grid=(GRID,),
        in_specs=[
            pl.BlockSpec((3, BLK), lambda i: (0, i)),
            pl.BlockSpec((MN, 3), lambda i: (0, 0)),
        ],
        out_specs=[
            pl.BlockSpec((MN // 32, BLK), lambda i: (0, i)),
            pl.BlockSpec(memory_space=pltpu.SMEM),
        ],
        out_shape=[
            jax.ShapeDtypeStruct((MN // 32, NPAD), jnp.int32),
            jax.ShapeDtypeStruct((1, 1), jnp.float32),
        ],
        compiler_params=pltpu.CompilerParams(
            dimension_semantics=("arbitrary",)),
    )(meansT, node_positions)

    pix, c0, c1, c2, wv = pl.pallas_call(
        _p2_body,
        grid=(GRID,),
        in_specs=[
            pl.BlockSpec((3, BLK), lambda i: (0, i)),
            pl.BlockSpec((MN // 32, BLK), lambda i: (0, i)),
            pl.BlockSpec(memory_space=pltpu.SMEM),
            pl.BlockSpec((MN, 3), lambda i: (0, 0)),
            pl.BlockSpec((MN, 3), lambda i: (0, 0)),
            pl.BlockSpec((3, BLK), lambda i: (0, i)),
            pl.BlockSpec((1, BLK), lambda i: (0, i)),
            pl.BlockSpec((1, BLK), lambda i: (0, i)),
            pl.BlockSpec((3, 3), lambda i: (0, 0)),
            pl.BlockSpec((4, 4), lambda i: (0, 0)),
        ],
        out_specs=[
            pl.BlockSpec((1, BLK), lambda i: (0, i)),
            pl.BlockSpec((1, BLK), lambda i: (0, i)),
            pl.BlockSpec((1, BLK), lambda i: (0, i)),
            pl.BlockSpec((1, BLK), lambda i: (0, i)),
            pl.BlockSpec((1, BLK), lambda i: (0, i)),
        ],
        out_shape=[
            jax.ShapeDtypeStruct((1, NPAD), jnp.int32),
            jax.ShapeDtypeStruct((1, NPAD), jnp.float32),
            jax.ShapeDtypeStruct((1, NPAD), jnp.float32),
            jax.ShapeDtypeStruct((1, NPAD), jnp.float32),
            jax.ShapeDtypeStruct((1, NPAD), jnp.float32),
        ],
        compiler_params=pltpu.CompilerParams(
            dimension_semantics=("arbitrary",)),
    )(meansT, packed, s8, node_positions, offsets_t, clogT, olog, lsc,
      intrinsics, world_to_camera)

    mesh = plsc.VectorSubcoreMesh(core_axis_name="c", subcore_axis_name="s")
    scatter = pl.kernel(
        _sc_body,
        out_type=jax.ShapeDtypeStruct((3, HWPIX), jnp.float32),
        mesh=mesh,
        compiler_params=pltpu.CompilerParams(needs_layout_passes=False),
        scratch_types=[
            pltpu.VMEM((1, CHUNK), jnp.int32),
            pltpu.VMEM((1, CHUNK), jnp.float32),
            pltpu.VMEM((1, CHUNK), jnp.float32),
            pltpu.VMEM((1, CHUNK), jnp.float32),
            pltpu.VMEM((1, CHUNK), jnp.float32),
            pltpu.VMEM((PPT,), jnp.float32),
            pltpu.VMEM((PPT,), jnp.float32),
            pltpu.VMEM((PPT,), jnp.float32),
            pltpu.VMEM((PPT,), jnp.float32),
            pltpu.VMEM((1, PPT), jnp.float32),
            pltpu.VMEM((1, PPT), jnp.float32),
            pltpu.VMEM((1, PPT), jnp.float32),
        ],
    )
    out = scatter(pix.reshape(NCHUNK, CHUNK), c0.reshape(NCHUNK, CHUNK),
                  c1.reshape(NCHUNK, CHUNK), c2.reshape(NCHUNK, CHUNK),
                  wv.reshape(NCHUNK, CHUNK))
    return jnp.transpose(out).reshape(HH, WW, 3)


# speedup vs baseline: 10.9600x; 1.0418x over previous
import functools

import jax
import jax.numpy as jnp
from jax import lax
from jax.experimental import pallas as pl
from jax.experimental.pallas import tpu as pltpu
from jax.experimental.pallas import tpu_sc as plsc

NG = 100000
MN = 512
KN = 8
HH = 512
WW = 512
HWPIX = HH * WW

BLK = 2048
NPAD = 100352
GRID = NPAD // BLK

NCORES = 2
NSUB = 16
NTILES = NCORES * NSUB
PPT = HWPIX // NTILES
CHUNK = 2048
NCHUNK = NPAD // CHUNK
LANES = 16


def _dist_block(meansT, nodes):
    mnorm = jnp.sum(meansT * meansT, axis=0, keepdims=True)
    nnorm = jnp.sum(nodes * nodes, axis=1, keepdims=True)
    dot = jnp.dot(nodes, meansT, preferred_element_type=jnp.float32)
    d2 = (mnorm + nnorm) - 2.0 * dot
    return jnp.maximum(jnp.sqrt(jnp.maximum(d2, 0.0)), 1e-6)


def _p1_body(meansT_ref, nodes_ref, packed_ref, sum_ref):
    pid = pl.program_id(0)
    dist = _dist_block(meansT_ref[...], nodes_ref[...])
    d = dist
    s8 = jnp.zeros((1, BLK), jnp.float32)
    riota = lax.broadcasted_iota(jnp.int32, (MN, BLK), 0)
    msk = jnp.zeros((MN, BLK), jnp.bool_)
    for _ in range(KN):
        m = jnp.min(d, axis=0, keepdims=True)
        s8 = s8 + m
        fidx = jnp.min(jnp.where(d == m, riota, MN), axis=0, keepdims=True)
        sel = riota == fidx
        msk = msk | sel
        d = jnp.where(sel, jnp.float32(jnp.inf), d)
    mi = msk.astype(jnp.int32)
    shifts = lax.broadcasted_iota(jnp.int32, (32, 1), 0)
    words = [jnp.sum(mi[w * 32:(w + 1) * 32, :] << shifts, axis=0, keepdims=True)
             for w in range(MN // 32)]
    packed_ref[...] = jnp.concatenate(words, axis=0)
    col = pid * BLK + lax.broadcasted_iota(jnp.int32, (1, BLK), 1)
    total = jnp.sum(jnp.where(col < NG, s8, 0.0))
    @pl.when(pid == 0)
    def _():
        sum_ref[0, 0] = total
    @pl.when(pid != 0)
    def _():
        sum_ref[0, 0] = sum_ref[0, 0] + total


def _p2_body(meansT_ref, packed_ref, sum_ref, nodes_ref, offT_ref, clogT_ref,
             olog_ref, lsc_ref, intr_ref, w2c_ref,
             pix_ref, c0_ref, c1_ref, c2_ref, wv_ref):
    pid = pl.program_id(0)
    meansT = meansT_ref[...]
    dist = _dist_block(meansT, nodes_ref[...])
    tau = sum_ref[0, 0] / jnp.float32(NG * KN) + 1e-6
    packed = packed_ref[...]
    shifts = lax.broadcasted_iota(jnp.int32, (32, 1), 0)
    parts = [(jnp.broadcast_to(packed[w:w + 1, :], (32, BLK)) >> shifts) & 1
             for w in range(MN // 32)]
    msk = jnp.concatenate(parts, axis=0) == 1
    v1 = jnp.min(jnp.where(msk, dist, jnp.float32(jnp.inf)),
                 axis=0, keepdims=True)
    e = jnp.where(msk, jnp.exp((v1 - dist) / tau), 0.0)
    wn = e / jnp.sum(e, axis=0, keepdims=True)
    off = offT_ref[...]
    mo0 = jnp.sum(wn * off[:, 0:1], axis=0, keepdims=True)
    mo1 = jnp.sum(wn * off[:, 1:2], axis=0, keepdims=True)
    mo2 = jnp.sum(wn * off[:, 2:3], axis=0, keepdims=True)
    mt = jnp.concatenate(
        [meansT[0:1, :] + mo0, meansT[1:2, :] + mo1, meansT[2:3, :] + mo2],
        axis=0)
    w2c = w2c_ref[...]
    R = w2c[0:3, 0:3]
    t = w2c[0:3, 3:4]
    pts = jnp.dot(R, mt, preferred_element_type=jnp.float32) + t
    uvw = jnp.dot(intr_ref[...], pts,
                  preferred_element_type=jnp.float32)
    z = jnp.maximum(uvw[2:3, :], 1e-3)
    u = uvw[0:1, :] / z
    v = uvw[1:2, :] / z
    ui = jnp.clip(jnp.round(u), 0.0, WW - 1).astype(jnp.int32)
    vi = jnp.clip(jnp.round(v), 0.0, HH - 1).astype(jnp.int32)
    pix = vi * WW + ui
    opac = jax.nn.sigmoid(olog_ref[...])
    scale = jnp.exp(lsc_ref[...])
    wgt = opac * scale / (z * z)
    col = pid * BLK + lax.broadcasted_iota(jnp.int32, (1, BLK), 1)
    valid = col < NG
    wgt = jnp.where(valid, wgt, 0.0)
    pix_ref[...] = jnp.where(valid, pix, 0)
    c = jax.nn.sigmoid(clogT_ref[...])
    c0_ref[...] = c[0:1, :] * wgt
    c1_ref[...] = c[1:2, :] * wgt
    c2_ref[...] = c[2:3, :] * wgt
    wv_ref[...] = wgt


def _sc_body(pix_hbm, c0_hbm, c1_hbm, c2_hbm, wv_hbm, out_hbm,
             pixb, c0b, c1b, c2b, wvb, acc0, acc1, acc2, accw,
             st0, st1, st2):
    wid = lax.axis_index("s") * NCORES + lax.axis_index("c")
    base = wid * PPT

    def zero(g, carry):
        s = g * LANES
        z = jnp.zeros((LANES,), jnp.float32)
        acc0[pl.ds(s, LANES)] = z
        acc1[pl.ds(s, LANES)] = z
        acc2[pl.ds(s, LANES)] = z
        accw[pl.ds(s, LANES)] = z
        return carry

    lax.fori_loop(0, PPT // LANES, zero, 0)

    def chunk_body(ci, carry):
        pltpu.sync_copy(pix_hbm.at[pl.ds(ci, 1), :], pixb)
        pltpu.sync_copy(c0_hbm.at[pl.ds(ci, 1), :], c0b)
        pltpu.sync_copy(c1_hbm.at[pl.ds(ci, 1), :], c1b)
        pltpu.sync_copy(c2_hbm.at[pl.ds(ci, 1), :], c2b)
        pltpu.sync_copy(wv_hbm.at[pl.ds(ci, 1), :], wvb)

        def grp(g, carry2):
            s = g * LANES
            pv = pixb[0, pl.ds(s, LANES)]
            lv = pv - base
            mk = (lv >= 0) & (lv < PPT)
            ls = jnp.where(mk, lv, 0)
            plsc.addupdate_scatter(acc0, [ls], c0b[0, pl.ds(s, LANES)], mask=mk)
            plsc.addupdate_scatter(acc1, [ls], c1b[0, pl.ds(s, LANES)], mask=mk)
            plsc.addupdate_scatter(acc2, [ls], c2b[0, pl.ds(s, LANES)], mask=mk)
            plsc.addupdate_scatter(accw, [ls], wvb[0, pl.ds(s, LANES)], mask=mk)
            return carry2

        return lax.fori_loop(0, CHUNK // LANES, grp, carry)

    lax.fori_loop(0, NCHUNK, chunk_body, 0)

    def norm(g, carry):
        s = g * LANES
        ws = accw[pl.ds(s, LANES)]
        alpha = jnp.clip(ws, 0.0, 1.0)
        sc = alpha / (ws + 1e-6)
        st0[0, pl.ds(s, LANES)] = acc0[pl.ds(s, LANES)] * sc
        st1[0, pl.ds(s, LANES)] = acc1[pl.ds(s, LANES)] * sc
        st2[0, pl.ds(s, LANES)] = acc2[pl.ds(s, LANES)] * sc
        return carry

    lax.fori_loop(0, PPT // LANES, norm, 0)
    pltpu.sync_copy(st0, out_hbm.at[pl.ds(0, 1), pl.ds(base, PPT)])
    pltpu.sync_copy(st1, out_hbm.at[pl.ds(1, 1), pl.ds(base, PPT)])
    pltpu.sync_copy(st2, out_hbm.at[pl.ds(2, 1), pl.ds(base, PPT)])


def kernel(means, log_scales, color_logits, opacity_logits, node_positions,
           node_offsets, intrinsics, world_to_camera, time_index):
    t = jnp.asarray(time_index)
    offsets_t = lax.dynamic_index_in_dim(node_offsets, t, 0, keepdims=False)

    pad = NPAD - NG
    meansT = jnp.pad(jnp.transpose(means), ((0, 0), (0, pad)))
    clogT = jnp.pad(jnp.transpose(color_logits), ((0, 0), (0, pad)))
    olog = jnp.pad(jnp.transpose(opacity_logits), ((0, 0), (0, pad)))
    lsc = jnp.pad(jnp.transpose(log_scales), ((0, 0), (0, pad)))

    packed, s8 = pl.pallas_call(
        _p1_body,
        grid=(GRID,),
        in_specs=[
            pl.BlockSpec((3, BLK), lambda i: (0, i)),
            pl.BlockSpec((MN, 3), lambda i: (0, 0)),
        ],
        out_specs=[
            pl.BlockSpec((MN // 32, BLK), lambda i: (0, i)),
            pl.BlockSpec(memory_space=pltpu.SMEM),
        ],
        out_shape=[
            jax.ShapeDtypeStruct((MN // 32, NPAD), jnp.int32),
            jax.ShapeDtypeStruct((1, 1), jnp.float32),
        ],
        compiler_params=pltpu.CompilerParams(
            dimension_semantics=("arbitrary",)),
    )(meansT, node_positions)

    pix, c0, c1, c2, wv = pl.pallas_call(
        _p2_body,
        grid=(GRID,),
        in_specs=[
            pl.BlockSpec((3, BLK), lambda i: (0, i)),
            pl.BlockSpec((MN // 32, BLK), lambda i: (0, i)),
            pl.BlockSpec(memory_space=pltpu.SMEM),
            pl.BlockSpec((MN, 3), lambda i: (0, 0)),
            pl.BlockSpec((MN, 3), lambda i: (0, 0)),
            pl.BlockSpec((3, BLK), lambda i: (0, i)),
            pl.BlockSpec((1, BLK), lambda i: (0, i)),
            pl.BlockSpec((1, BLK), lambda i: (0, i)),
            pl.BlockSpec((3, 3), lambda i: (0, 0)),
            pl.BlockSpec((4, 4), lambda i: (0, 0)),
        ],
        out_specs=[
            pl.BlockSpec((1, BLK), lambda i: (0, i)),
            pl.BlockSpec((1, BLK), lambda i: (0, i)),
            pl.BlockSpec((1, BLK), lambda i: (0, i)),
            pl.BlockSpec((1, BLK), lambda i: (0, i)),
            pl.BlockSpec((1, BLK), lambda i: (0, i)),
        ],
        out_shape=[
            jax.ShapeDtypeStruct((1, NPAD), jnp.int32),
            jax.ShapeDtypeStruct((1, NPAD), jnp.float32),
            jax.ShapeDtypeStruct((1, NPAD), jnp.float32),
            jax.ShapeDtypeStruct((1, NPAD), jnp.float32),
            jax.ShapeDtypeStruct((1, NPAD), jnp.float32),
        ],
        compiler_params=pltpu.CompilerParams(
            dimension_semantics=("arbitrary",)),
    )(meansT, packed, s8, node_positions, offsets_t, clogT, olog, lsc,
      intrinsics, world_to_camera)

    mesh = plsc.VectorSubcoreMesh(core_axis_name="c", subcore_axis_name="s")
    scatter = pl.kernel(
        _sc_body,
        out_type=jax.ShapeDtypeStruct((3, HWPIX), jnp.float32),
        mesh=mesh,
        compiler_params=pltpu.CompilerParams(needs_layout_passes=False),
        scratch_types=[
            pltpu.VMEM((1, CHUNK), jnp.int32),
            pltpu.VMEM((1, CHUNK), jnp.float32),
            pltpu.VMEM((1, CHUNK), jnp.float32),
            pltpu.VMEM((1, CHUNK), jnp.float32),
            pltpu.VMEM((1, CHUNK), jnp.float32),
            pltpu.VMEM((PPT,), jnp.float32),
            pltpu.VMEM((PPT,), jnp.float32),
            pltpu.VMEM((PPT,), jnp.float32),
            pltpu.VMEM((PPT,), jnp.float32),
            pltpu.VMEM((1, PPT), jnp.float32),
            pltpu.VMEM((1, PPT), jnp.float32),
            pltpu.VMEM((1, PPT), jnp.float32),
        ],
    )
    out = scatter(pix.reshape(NCHUNK, CHUNK), c0.reshape(NCHUNK, CHUNK),
                  c1.reshape(NCHUNK, CHUNK), c2.reshape(NCHUNK, CHUNK),
                  wv.reshape(NCHUNK, CHUNK))
    return jnp.transpose(out).reshape(HH, WW, 3)


# speedup vs baseline: 11.9346x; 1.0889x over previous
import functools

import jax
import jax.numpy as jnp
from jax import lax
from jax.experimental import pallas as pl
from jax.experimental.pallas import tpu as pltpu
from jax.experimental.pallas import tpu_sc as plsc

NG = 100000
MN = 512
KN = 8
HH = 512
WW = 512
HWPIX = HH * WW

BLK = 2048
NPAD = 100352
GRID = NPAD // BLK

NCORES = 2
NSUB = 16
NTILES = NCORES * NSUB
PPT = HWPIX // NTILES
CHUNK = 2048
NCHUNK = NPAD // CHUNK
LANES = 16


def _dist_block(meansT, nodes):
    mnorm = jnp.sum(meansT * meansT, axis=0, keepdims=True)
    nnorm = jnp.sum(nodes * nodes, axis=1, keepdims=True)
    dot = jnp.dot(nodes, meansT, preferred_element_type=jnp.float32)
    d2 = (mnorm + nnorm) - 2.0 * dot
    return jnp.maximum(jnp.sqrt(jnp.maximum(d2, 0.0)), 1e-6)


def _p1_body(meansT_ref, nodes_ref, packed_ref, sum_ref):
    pid = pl.program_id(0)
    dist = _dist_block(meansT_ref[...], nodes_ref[...])
    d = dist
    s8 = jnp.zeros((1, BLK), jnp.float32)
    riota = lax.broadcasted_iota(jnp.int32, (MN, BLK), 0)
    msk = jnp.zeros((MN, BLK), jnp.bool_)
    for _ in range(KN):
        m = jnp.min(d, axis=0, keepdims=True)
        s8 = s8 + m
        fidx = jnp.min(jnp.where(d == m, riota, MN), axis=0, keepdims=True)
        sel = riota == fidx
        msk = msk | sel
        d = jnp.where(sel, jnp.float32(jnp.inf), d)
    mi = msk.astype(jnp.int32)
    shifts = lax.broadcasted_iota(jnp.int32, (32, 1), 0)
    words = [jnp.sum(mi[w * 32:(w + 1) * 32, :] << shifts, axis=0, keepdims=True)
             for w in range(MN // 32)]
    packed_ref[...] = jnp.concatenate(words, axis=0)
    col = pid * BLK + lax.broadcasted_iota(jnp.int32, (1, BLK), 1)
    total = jnp.sum(jnp.where(col < NG, s8, 0.0))
    @pl.when(pid == 0)
    def _():
        sum_ref[0, 0] = total
    @pl.when(pid != 0)
    def _():
        sum_ref[0, 0] = sum_ref[0, 0] + total


def _p2_body(meansT_ref, packed_ref, sum_ref, nodes_ref, offT_ref, clogT_ref,
             olog_ref, lsc_ref, intr_ref, w2c_ref,
             pix_ref, c0_ref, c1_ref, c2_ref, wv_ref):
    pid = pl.program_id(0)
    meansT = meansT_ref[...]
    dist = _dist_block(meansT, nodes_ref[...])
    tau = sum_ref[0, 0] / jnp.float32(NG * KN) + 1e-6
    packed = packed_ref[...]
    shifts = lax.broadcasted_iota(jnp.int32, (32, 1), 0)
    parts = [(jnp.broadcast_to(packed[w:w + 1, :], (32, BLK)) >> shifts) & 1
             for w in range(MN // 32)]
    msk = jnp.concatenate(parts, axis=0) == 1
    v1 = jnp.min(jnp.where(msk, dist, jnp.float32(jnp.inf)),
                 axis=0, keepdims=True)
    e = jnp.where(msk, jnp.exp((v1 - dist) / tau), 0.0)
    wn = e / jnp.sum(e, axis=0, keepdims=True)
    off = offT_ref[...]
    mo0 = jnp.sum(wn * off[:, 0:1], axis=0, keepdims=True)
    mo1 = jnp.sum(wn * off[:, 1:2], axis=0, keepdims=True)
    mo2 = jnp.sum(wn * off[:, 2:3], axis=0, keepdims=True)
    mt = jnp.concatenate(
        [meansT[0:1, :] + mo0, meansT[1:2, :] + mo1, meansT[2:3, :] + mo2],
        axis=0)
    w2c = w2c_ref[...]
    R = w2c[0:3, 0:3]
    t = w2c[0:3, 3:4]
    pts = jnp.dot(R, mt, preferred_element_type=jnp.float32) + t
    uvw = jnp.dot(intr_ref[...], pts,
                  preferred_element_type=jnp.float32)
    z = jnp.maximum(uvw[2:3, :], 1e-3)
    u = uvw[0:1, :] / z
    v = uvw[1:2, :] / z
    ui = jnp.clip(jnp.round(u), 0.0, WW - 1).astype(jnp.int32)
    vi = jnp.clip(jnp.round(v), 0.0, HH - 1).astype(jnp.int32)
    pix = vi * WW + ui
    opac = jax.nn.sigmoid(olog_ref[...])
    scale = jnp.exp(lsc_ref[...])
    wgt = opac * scale / (z * z)
    col = pid * BLK + lax.broadcasted_iota(jnp.int32, (1, BLK), 1)
    valid = col < NG
    wgt = jnp.where(valid, wgt, 0.0)
    pix_ref[...] = jnp.where(valid, pix, 0)
    c = jax.nn.sigmoid(clogT_ref[...])
    c0_ref[...] = c[0:1, :] * wgt
    c1_ref[...] = c[1:2, :] * wgt
    c2_ref[...] = c[2:3, :] * wgt
    wv_ref[...] = wgt


def _sc_body(pix_hbm, c0_hbm, c1_hbm, c2_hbm, wv_hbm, out_hbm,
             pixb, c0b, c1b, c2b, wvb, acc0, acc1, acc2, accw,
             st0, st1, st2, dsem):
    wid = lax.axis_index("s") * NCORES + lax.axis_index("c")
    base = wid * PPT

    def zero(g, carry):
        s = g * LANES
        z = jnp.zeros((LANES,), jnp.float32)
        acc0[pl.ds(s, LANES)] = z
        acc1[pl.ds(s, LANES)] = z
        acc2[pl.ds(s, LANES)] = z
        accw[pl.ds(s, LANES)] = z
        return carry

    lax.fori_loop(0, PPT // LANES, zero, 0)

    def chunk_body(ci, carry):
        handles = [
            pltpu.async_copy(pix_hbm.at[pl.ds(ci, 1), :], pixb, dsem),
            pltpu.async_copy(c0_hbm.at[pl.ds(ci, 1), :], c0b, dsem),
            pltpu.async_copy(c1_hbm.at[pl.ds(ci, 1), :], c1b, dsem),
            pltpu.async_copy(c2_hbm.at[pl.ds(ci, 1), :], c2b, dsem),
            pltpu.async_copy(wv_hbm.at[pl.ds(ci, 1), :], wvb, dsem),
        ]
        for h in handles:
            h.wait()

        def grp(g, carry2):
            s = g * LANES
            pv = pixb[0, pl.ds(s, LANES)]
            lv = pv - base
            mk = (lv >= 0) & (lv < PPT)
            ls = jnp.where(mk, lv, 0)
            plsc.addupdate_scatter(acc0, [ls], c0b[0, pl.ds(s, LANES)], mask=mk)
            plsc.addupdate_scatter(acc1, [ls], c1b[0, pl.ds(s, LANES)], mask=mk)
            plsc.addupdate_scatter(acc2, [ls], c2b[0, pl.ds(s, LANES)], mask=mk)
            plsc.addupdate_scatter(accw, [ls], wvb[0, pl.ds(s, LANES)], mask=mk)
            return carry2

        return lax.fori_loop(0, CHUNK // LANES, grp, carry)

    lax.fori_loop(0, NCHUNK, chunk_body, 0)

    def norm(g, carry):
        s = g * LANES
        ws = accw[pl.ds(s, LANES)]
        alpha = jnp.clip(ws, 0.0, 1.0)
        sc = alpha / (ws + 1e-6)
        st0[0, pl.ds(s, LANES)] = acc0[pl.ds(s, LANES)] * sc
        st1[0, pl.ds(s, LANES)] = acc1[pl.ds(s, LANES)] * sc
        st2[0, pl.ds(s, LANES)] = acc2[pl.ds(s, LANES)] * sc
        return carry

    lax.fori_loop(0, PPT // LANES, norm, 0)
    pltpu.sync_copy(st0, out_hbm.at[pl.ds(0, 1), pl.ds(base, PPT)])
    pltpu.sync_copy(st1, out_hbm.at[pl.ds(1, 1), pl.ds(base, PPT)])
    pltpu.sync_copy(st2, out_hbm.at[pl.ds(2, 1), pl.ds(base, PPT)])


def kernel(means, log_scales, color_logits, opacity_logits, node_positions,
           node_offsets, intrinsics, world_to_camera, time_index):
    t = jnp.asarray(time_index)
    offsets_t = lax.dynamic_index_in_dim(node_offsets, t, 0, keepdims=False)

    pad = NPAD - NG
    meansT = jnp.pad(jnp.transpose(means), ((0, 0), (0, pad)))
    clogT = jnp.pad(jnp.transpose(color_logits), ((0, 0), (0, pad)))
    olog = jnp.pad(jnp.transpose(opacity_logits), ((0, 0), (0, pad)))
    lsc = jnp.pad(jnp.transpose(log_scales), ((0, 0), (0, pad)))

    packed, s8 = pl.pallas_call(
        _p1_body,
        grid=(GRID,),
        in_specs=[
            pl.BlockSpec((3, BLK), lambda i: (0, i)),
            pl.BlockSpec((MN, 3), lambda i: (0, 0)),
        ],
        out_specs=[
            pl.BlockSpec((MN // 32, BLK), lambda i: (0, i)),
            pl.BlockSpec(memory_space=pltpu.SMEM),
        ],
        out_shape=[
            jax.ShapeDtypeStruct((MN // 32, NPAD), jnp.int32),
            jax.ShapeDtypeStruct((1, 1), jnp.float32),
        ],
        compiler_params=pltpu.CompilerParams(
            dimension_semantics=("arbitrary",)),
    )(meansT, node_positions)

    pix, c0, c1, c2, wv = pl.pallas_call(
        _p2_body,
        grid=(GRID,),
        in_specs=[
            pl.BlockSpec((3, BLK), lambda i: (0, i)),
            pl.BlockSpec((MN // 32, BLK), lambda i: (0, i)),
            pl.BlockSpec(memory_space=pltpu.SMEM),
            pl.BlockSpec((MN, 3), lambda i: (0, 0)),
            pl.BlockSpec((MN, 3), lambda i: (0, 0)),
            pl.BlockSpec((3, BLK), lambda i: (0, i)),
            pl.BlockSpec((1, BLK), lambda i: (0, i)),
            pl.BlockSpec((1, BLK), lambda i: (0, i)),
            pl.BlockSpec((3, 3), lambda i: (0, 0)),
            pl.BlockSpec((4, 4), lambda i: (0, 0)),
        ],
        out_specs=[
            pl.BlockSpec((1, BLK), lambda i: (0, i)),
            pl.BlockSpec((1, BLK), lambda i: (0, i)),
            pl.BlockSpec((1, BLK), lambda i: (0, i)),
            pl.BlockSpec((1, BLK), lambda i: (0, i)),
            pl.BlockSpec((1, BLK), lambda i: (0, i)),
        ],
        out_shape=[
            jax.ShapeDtypeStruct((1, NPAD), jnp.int32),
            jax.ShapeDtypeStruct((1, NPAD), jnp.float32),
            jax.ShapeDtypeStruct((1, NPAD), jnp.float32),
            jax.ShapeDtypeStruct((1, NPAD), jnp.float32),
            jax.ShapeDtypeStruct((1, NPAD), jnp.float32),
        ],
        compiler_params=pltpu.CompilerParams(
            dimension_semantics=("arbitrary",)),
    )(meansT, packed, s8, node_positions, offsets_t, clogT, olog, lsc,
      intrinsics, world_to_camera)

    mesh = plsc.VectorSubcoreMesh(core_axis_name="c", subcore_axis_name="s")
    scatter = pl.kernel(
        _sc_body,
        out_type=jax.ShapeDtypeStruct((3, HWPIX), jnp.float32),
        mesh=mesh,
        compiler_params=pltpu.CompilerParams(needs_layout_passes=False),
        scratch_types=[
            pltpu.VMEM((1, CHUNK), jnp.int32),
            pltpu.VMEM((1, CHUNK), jnp.float32),
            pltpu.VMEM((1, CHUNK), jnp.float32),
            pltpu.VMEM((1, CHUNK), jnp.float32),
            pltpu.VMEM((1, CHUNK), jnp.float32),
            pltpu.VMEM((PPT,), jnp.float32),
            pltpu.VMEM((PPT,), jnp.float32),
            pltpu.VMEM((PPT,), jnp.float32),
            pltpu.VMEM((PPT,), jnp.float32),
            pltpu.VMEM((1, PPT), jnp.float32),
            pltpu.VMEM((1, PPT), jnp.float32),
            pltpu.VMEM((1, PPT), jnp.float32),
            pltpu.SemaphoreType.DMA,
        ],
    )
    out = scatter(pix.reshape(NCHUNK, CHUNK), c0.reshape(NCHUNK, CHUNK),
                  c1.reshape(NCHUNK, CHUNK), c2.reshape(NCHUNK, CHUNK),
                  wv.reshape(NCHUNK, CHUNK))
    return jnp.transpose(out).reshape(HH, WW, 3)
